# Initial kernel scaffold; baseline (speedup 1.0000x reference)
import jax
import jax.numpy as jnp
from jax.experimental import pallas as pl


def kernel(src_h, edge_index, W, b):
    raise NotImplementedError("write your pallas kernel here")


# speedup vs baseline: 4.6862x; 4.6862x over previous
import functools

import jax
import jax.numpy as jnp
from jax import lax
from jax.experimental import pallas as pl
from jax.experimental.pallas import tpu as pltpu
from jax.experimental.pallas import tpu_sc as plsc

N_NODES = 10000
N_EDGES = 320000
D = 128

NC = 2
NS = 16
NW = NC * NS
CHUNK = 128
N_CHUNKS = -(-N_EDGES // (NW * CHUNK))
P_PER_W = N_CHUNKS * CHUNK
ACC_ROWS = 10240
ZROWS = ACC_ROWS // NS
OROWS = ACC_ROWS // NS


def _sc_gather_scatter(src_h, src_idx, dst_idx):
    mesh = plsc.VectorSubcoreMesh(core_axis_name="c", subcore_axis_name="s")

    @functools.partial(
        pl.kernel,
        out_type=jax.ShapeDtypeStruct((NC, ACC_ROWS, D), jnp.float32),
        mesh=mesh,
        scratch_types=[
            pltpu.VMEM((N_CHUNKS, CHUNK), jnp.int32),
            pltpu.VMEM((N_CHUNKS, CHUNK), jnp.int32),
            pltpu.VMEM((CHUNK, D), jnp.float32),
            pltpu.VMEM_SHARED((ACC_ROWS, D), jnp.float32),
            pltpu.SemaphoreType.DMA,
        ],
    )
    def k(h_hbm, src_hbm, dst_hbm, out_hbm, src_v, dst_v, rows_v, acc, sem):
        c = lax.axis_index("c")
        s = lax.axis_index("s")

        pltpu.sync_copy(src_hbm.at[c, s], src_v)
        pltpu.sync_copy(dst_hbm.at[c, s], dst_v)

        def zrow(i, carry):
            for j in range(D // 16):
                rows_v[i, pl.ds(j * 16, 16)] = jnp.zeros((16,), jnp.float32)
            return carry
        lax.fori_loop(0, CHUNK, zrow, 0)
        zbase = s * ZROWS
        for t in range(ZROWS // CHUNK):
            pltpu.sync_copy(rows_v, acc.at[pl.ds(zbase + t * CHUNK, CHUNK)])
        rem = ZROWS % CHUNK
        if rem:
            pltpu.sync_copy(rows_v.at[pl.ds(0, rem)],
                            acc.at[pl.ds(zbase + (ZROWS // CHUNK) * CHUNK, rem)])
        plsc.subcore_barrier()

        def body(j, carry):
            pltpu.async_copy(h_hbm.at[src_v.at[j]], rows_v, sem).wait()
            pltpu.sync_copy(rows_v, acc.at[dst_v.at[j]], add=True)
            return carry
        lax.fori_loop(0, N_CHUNKS, body, 0)
        plsc.subcore_barrier()

        obase = s * OROWS
        pltpu.sync_copy(acc.at[pl.ds(obase, OROWS)],
                        out_hbm.at[c].at[pl.ds(obase, OROWS)])

    return k(src_h, src_idx, dst_idx)


def _tc_linear(acc2, W, b2):
    BR = 2000

    def body(a0_ref, a1_ref, w_ref, b_ref, o_ref):
        x = a0_ref[...] + a1_ref[...]
        o_ref[...] = lax.dot_general(
            x, w_ref[...], (((1,), (1,)), ((), ())),
            preferred_element_type=jnp.float32) + b_ref[...]

    return pl.pallas_call(
        body,
        grid=(N_NODES // BR,),
        in_specs=[
            pl.BlockSpec((BR, D), lambda i: (i, 0)),
            pl.BlockSpec((BR, D), lambda i: (i, 0)),
            pl.BlockSpec((D, D), lambda i: (0, 0)),
            pl.BlockSpec((1, D), lambda i: (0, 0)),
        ],
        out_specs=pl.BlockSpec((BR, D), lambda i: (i, 0)),
        out_shape=jax.ShapeDtypeStruct((N_NODES, D), jnp.float32),
    )(acc2[0], acc2[1], W, b2)


def kernel(src_h, edge_index, W, b):
    pad = NW * P_PER_W - N_EDGES
    src = jnp.concatenate([edge_index[0], jnp.zeros((pad,), jnp.int32)])
    dst = jnp.concatenate([edge_index[1], jnp.full((pad,), N_NODES, jnp.int32)])
    src_idx = src.reshape(NC, NS, N_CHUNKS, CHUNK)
    dst_idx = dst.reshape(NC, NS, N_CHUNKS, CHUNK)
    acc2 = _sc_gather_scatter(src_h, src_idx, dst_idx)
    return _tc_linear(acc2[:, :N_NODES], W, b.reshape(1, D))
